# baseline (speedup 1.0000x reference)
import functools

import jax
import jax.numpy as jnp
from jax import lax
from jax.experimental import pallas as pl
from jax.experimental.pallas import tpu as pltpu
from jax.experimental.pallas import tpu_sc as plsc

N_REC = 3
D = 768
NW = 32
TILE = 32
TB = 8
TOK = 2048
K_SC = 5


def _sc_router(xf, W, nsc):
    tps = nsc // NW
    nt = tps // TILE
    mesh = plsc.VectorSubcoreMesh(core_axis_name="c", subcore_axis_name="s")

    @functools.partial(
        pl.kernel,
        out_type=[jax.ShapeDtypeStruct((nsc,), jnp.float32)] * 4,
        mesh=mesh,
        scratch_types=[
            pltpu.VMEM((N_REC, D), jnp.float32),
            pltpu.VMEM((TILE, D), jnp.float32),
            pltpu.VMEM((TILE, D), jnp.float32),
            pltpu.VMEM((tps,), jnp.float32),
            pltpu.VMEM((tps,), jnp.float32),
            pltpu.VMEM((tps,), jnp.float32),
            pltpu.VMEM((tps,), jnp.float32),
            pltpu.SemaphoreType.DMA,
            pltpu.SemaphoreType.DMA,
        ],
        compiler_params=pltpu.CompilerParams(
            use_tc_tiling_on_sc=True, needs_layout_passes=False),
    )
    def k(x_hbm, w_hbm, o0_hbm, o1_hbm, o2_hbm, og_hbm,
          w_v, xb0, xb1, l0_v, l1_v, l2_v, gt_v, sem0, sem1):
        wid = lax.axis_index("s") * 2 + lax.axis_index("c")
        base = wid * tps
        pltpu.sync_copy(w_hbm, w_v)
        iota = lax.broadcasted_iota(jnp.int32, (16,), 0)

        def start_copy(t, buf, sem):
            pltpu.async_copy(
                x_hbm.at[pl.ds(base + t * TILE, TILE), :], buf, sem)

        def wait_copy(buf, sem):
            pltpu.make_async_copy(
                x_hbm.at[pl.ds(base, TILE), :], buf, sem).wait()

        def compute_tile(t, buf):
            zeros16 = jnp.zeros((16,), jnp.float32)

            @pl.loop(0, TILE // 16)
            def _(g):
                logit_vecs = [zeros16, zeros16, zeros16]
                for blk in range(16 // TB):

                    @pl.loop(0, D // 16,
                             init_carry=(zeros16,) * (3 * TB), unroll=2)
                    def accs(c, carry):
                        acc = list(carry)
                        c16 = c * 16
                        w0 = w_v[0, pl.ds(c16, 16)]
                        w1 = w_v[1, pl.ds(c16, 16)]
                        w2 = w_v[2, pl.ds(c16, 16)]
                        tokb = g * 16 + blk * TB
                        for tt in range(TB):
                            xv = buf[tokb + tt, pl.ds(c16, 16)]
                            acc[3 * tt] = acc[3 * tt] + xv * w0
                            acc[3 * tt + 1] = acc[3 * tt + 1] + xv * w1
                            acc[3 * tt + 2] = acc[3 * tt + 2] + xv * w2
                        return tuple(acc)

                    for tt in range(TB):
                        mask = iota == (blk * TB + tt)
                        for n in range(3):
                            s = jnp.sum(accs[3 * tt + n])
                            logit_vecs[n] = jnp.where(
                                mask, s, logit_vecs[n])

                l0, l1, l2 = logit_vecs
                m = jnp.maximum(jnp.maximum(l0, l1), l2)
                ssum = jnp.exp(l0 - m) + jnp.exp(l1 - m) + jnp.exp(l2 - m)
                off = t * TILE + g * 16
                gt_v[pl.ds(off, 16)] = 1.0 / ssum
                l0_v[pl.ds(off, 16)] = l0
                l1_v[pl.ds(off, 16)] = l1
                l2_v[pl.ds(off, 16)] = l2

        start_copy(0, xb0, sem0)
        start_copy(1, xb1, sem1)

        @pl.loop(0, nt, step=2)
        def _(t):
            wait_copy(xb0, sem0)
            compute_tile(t, xb0)

            @pl.when(t + 2 < nt)
            def _():
                start_copy(t + 2, xb0, sem0)

            wait_copy(xb1, sem1)
            compute_tile(t + 1, xb1)

            @pl.when(t + 3 < nt)
            def _():
                start_copy(t + 3, xb1, sem1)

        pltpu.sync_copy(l0_v, o0_hbm.at[pl.ds(base, tps)])
        pltpu.sync_copy(l1_v, o1_hbm.at[pl.ds(base, tps)])
        pltpu.sync_copy(l2_v, o2_hbm.at[pl.ds(base, tps)])
        pltpu.sync_copy(gt_v, og_hbm.at[pl.ds(base, tps)])

    return k(xf, W)


def _tc_router(xf, W, B, T):
    N = xf.shape[0]
    grid = N // TOK
    tpb = T // TOK

    def body(xa_ref, xb_ref, w_ref, gate_ref, logits_ref):
        b = pl.program_id(0)
        j = pl.program_id(1)

        @pl.when(b * tpb + j >= K_SC)
        def _():
            dn = (((1,), (1,)), ((), ()))
            logits = (
                lax.dot_general(w_ref[:, :D // 2], xa_ref[...], dn,
                                preferred_element_type=jnp.float32)
                + lax.dot_general(w_ref[:, D // 2:], xb_ref[...], dn,
                                  preferred_element_type=jnp.float32)
            )
            m = jnp.max(logits, axis=0, keepdims=True)
            s = jnp.sum(jnp.exp(logits - m), axis=0, keepdims=True)
            logits_ref[:, 0, pl.ds(j, 1), :] = logits[:, None, :]
            gate_ref[0, pl.ds(j, 1), :] = (1.0 / s)

    skip = lambda b, j: jnp.where(b * tpb + j < K_SC, K_SC, b * tpb + j)
    return pl.pallas_call(
        body,
        grid=(B, tpb),
        in_specs=[
            pl.BlockSpec((TOK, D // 2), lambda b, j: (skip(b, j), 0)),
            pl.BlockSpec((TOK, D // 2), lambda b, j: (skip(b, j), 1)),
            pl.BlockSpec((N_REC, D), lambda b, j: (0, 0)),
        ],
        out_specs=[
            pl.BlockSpec((1, tpb, TOK), lambda b, j: (b, 0, 0)),
            pl.BlockSpec((N_REC, 1, tpb, TOK),
                         lambda b, j: (0, b, 0, 0)),
        ],
        out_shape=[
            jax.ShapeDtypeStruct((B, tpb, TOK), jnp.float32),
            jax.ShapeDtypeStruct((N_REC, B, tpb, TOK), jnp.float32),
        ],
        compiler_params=pltpu.CompilerParams(
            dimension_semantics=("arbitrary", "arbitrary")),
    )(xf, xf, W)


def _stitch(gate, logits, l0, l1, l2, gt, B, T):
    tpb = T // TOK

    def body(g_in, lg_in, l0_ref, l1_ref, l2_ref, gt_ref,
             gate_ref, logits_ref):
        c = pl.program_id(0)
        j = lax.rem(c, tpb)

        @pl.when(j == 0)
        def _():
            gate_ref[...] = g_in[...]
            logits_ref[...] = lg_in[...]

        gate_ref[0, pl.ds(j, 1), :] = gt_ref[...].reshape(1, TOK)
        logits_ref[:, 0, pl.ds(j, 1), :] = jnp.concatenate(
            [l0_ref[...].reshape(1, 1, TOK),
             l1_ref[...].reshape(1, 1, TOK),
             l2_ref[...].reshape(1, 1, TOK)], axis=0)

    return pl.pallas_call(
        body,
        grid=(K_SC,),
        in_specs=[
            pl.BlockSpec((1, tpb, TOK), lambda c: (c // tpb, 0, 0)),
            pl.BlockSpec((N_REC, 1, tpb, TOK),
                         lambda c: (0, c // tpb, 0, 0)),
            pl.BlockSpec((TOK,), lambda c: (c,)),
            pl.BlockSpec((TOK,), lambda c: (c,)),
            pl.BlockSpec((TOK,), lambda c: (c,)),
            pl.BlockSpec((TOK,), lambda c: (c,)),
        ],
        out_specs=[
            pl.BlockSpec((1, tpb, TOK), lambda c: (c // tpb, 0, 0)),
            pl.BlockSpec((N_REC, 1, tpb, TOK),
                         lambda c: (0, c // tpb, 0, 0)),
        ],
        out_shape=[
            jax.ShapeDtypeStruct((B, tpb, TOK), jnp.float32),
            jax.ShapeDtypeStruct((N_REC, B, tpb, TOK), jnp.float32),
        ],
        input_output_aliases={0: 0, 1: 1},
        compiler_params=pltpu.CompilerParams(
            dimension_semantics=("arbitrary",)),
    )(gate, logits, l0, l1, l2, gt)


def kernel(x, W):
    B, T, _ = x.shape
    N = B * T
    nsc = K_SC * TOK
    xf = x.reshape(N, D)
    l0, l1, l2, gt = _sc_router(xf, W, nsc)
    gate, logits_t = _tc_router(xf, W, B, T)
    gate, logits_t = _stitch(gate, logits_t, l0, l1, l2, gt, B, T)
    selected = jnp.broadcast_to(
        jnp.arange(T, dtype=jnp.int32)[None, :, None], (B, T, 1))
    return (selected, gate.reshape(B, T, 1),
            jnp.transpose(logits_t.reshape(N_REC, B, T), (1, 2, 0)))


# speedup vs baseline: 1.0014x; 1.0014x over previous
import functools

import jax
import jax.numpy as jnp
from jax import lax
from jax.experimental import pallas as pl
from jax.experimental.pallas import tpu as pltpu
from jax.experimental.pallas import tpu_sc as plsc

N_REC = 3
D = 768
NW = 32
TILE = 32
TB = 8
TOK = 2048
K_SC = 5


def _sc_router(xf, W, nsc):
    tps = nsc // NW
    nt = tps // TILE
    mesh = plsc.VectorSubcoreMesh(core_axis_name="c", subcore_axis_name="s")

    @functools.partial(
        pl.kernel,
        out_type=[jax.ShapeDtypeStruct((nsc,), jnp.float32)] * 4,
        mesh=mesh,
        scratch_types=[
            pltpu.VMEM((N_REC, D), jnp.float32),
            pltpu.VMEM((TILE, D), jnp.float32),
            pltpu.VMEM((TILE, D), jnp.float32),
            pltpu.VMEM((tps,), jnp.float32),
            pltpu.VMEM((tps,), jnp.float32),
            pltpu.VMEM((tps,), jnp.float32),
            pltpu.VMEM((tps,), jnp.float32),
            pltpu.SemaphoreType.DMA,
            pltpu.SemaphoreType.DMA,
        ],
        compiler_params=pltpu.CompilerParams(
            use_tc_tiling_on_sc=True, needs_layout_passes=False),
    )
    def k(x_hbm, w_hbm, o0_hbm, o1_hbm, o2_hbm, og_hbm,
          w_v, xb0, xb1, l0_v, l1_v, l2_v, gt_v, sem0, sem1):
        wid = lax.axis_index("s") * 2 + lax.axis_index("c")
        base = wid * tps
        pltpu.sync_copy(w_hbm, w_v)
        iota = lax.broadcasted_iota(jnp.int32, (16,), 0)

        def start_copy(t, buf, sem):
            pltpu.async_copy(
                x_hbm.at[pl.ds(base + t * TILE, TILE), :], buf, sem)

        def wait_copy(buf, sem):
            pltpu.make_async_copy(
                x_hbm.at[pl.ds(base, TILE), :], buf, sem).wait()

        def compute_tile(t, buf):
            zeros16 = jnp.zeros((16,), jnp.float32)

            @pl.loop(0, TILE // 16)
            def _(g):
                logit_vecs = [zeros16, zeros16, zeros16]
                for blk in range(16 // TB):

                    @pl.loop(0, D // 16,
                             init_carry=(zeros16,) * (3 * TB), unroll=2)
                    def accs(c, carry):
                        acc = list(carry)
                        c16 = c * 16
                        w0 = w_v[0, pl.ds(c16, 16)]
                        w1 = w_v[1, pl.ds(c16, 16)]
                        w2 = w_v[2, pl.ds(c16, 16)]
                        tokb = g * 16 + blk * TB
                        for tt in range(TB):
                            xv = buf[tokb + tt, pl.ds(c16, 16)]
                            acc[3 * tt] = acc[3 * tt] + xv * w0
                            acc[3 * tt + 1] = acc[3 * tt + 1] + xv * w1
                            acc[3 * tt + 2] = acc[3 * tt + 2] + xv * w2
                        return tuple(acc)

                    for tt in range(TB):
                        mask = iota == (blk * TB + tt)
                        for n in range(3):
                            s = jnp.sum(accs[3 * tt + n])
                            logit_vecs[n] = jnp.where(
                                mask, s, logit_vecs[n])

                l0, l1, l2 = logit_vecs
                m = jnp.maximum(jnp.maximum(l0, l1), l2)
                ssum = jnp.exp(l0 - m) + jnp.exp(l1 - m) + jnp.exp(l2 - m)
                off = t * TILE + g * 16
                gt_v[pl.ds(off, 16)] = 1.0 / ssum
                l0_v[pl.ds(off, 16)] = l0
                l1_v[pl.ds(off, 16)] = l1
                l2_v[pl.ds(off, 16)] = l2

        start_copy(0, xb0, sem0)
        start_copy(1, xb1, sem1)

        @pl.loop(0, nt, step=2)
        def _(t):
            wait_copy(xb0, sem0)
            compute_tile(t, xb0)

            @pl.when(t + 2 < nt)
            def _():
                start_copy(t + 2, xb0, sem0)

            wait_copy(xb1, sem1)
            compute_tile(t + 1, xb1)

            @pl.when(t + 3 < nt)
            def _():
                start_copy(t + 3, xb1, sem1)

        pltpu.sync_copy(l0_v, o0_hbm.at[pl.ds(base, tps)])
        pltpu.sync_copy(l1_v, o1_hbm.at[pl.ds(base, tps)])
        pltpu.sync_copy(l2_v, o2_hbm.at[pl.ds(base, tps)])
        pltpu.sync_copy(gt_v, og_hbm.at[pl.ds(base, tps)])

    return k(xf, W)


def _tc_router(xf, W, B, T):
    N = xf.shape[0]
    grid = N // TOK
    tpb = T // TOK

    def body(x_ref, w_ref, gate_ref, logits_ref):
        b = pl.program_id(0)
        j = pl.program_id(1)

        @pl.when(b * tpb + j >= K_SC)
        def _():
            logits = lax.dot_general(
                w_ref[...], x_ref[...], (((1,), (1,)), ((), ())),
                preferred_element_type=jnp.float32)
            m = jnp.max(logits, axis=0, keepdims=True)
            s = jnp.sum(jnp.exp(logits - m), axis=0, keepdims=True)
            logits_ref[:, 0, pl.ds(j, 1), :] = logits[:, None, :]
            gate_ref[0, pl.ds(j, 1), :] = (1.0 / s)

    return pl.pallas_call(
        body,
        grid=(B, tpb),
        in_specs=[
            pl.BlockSpec(
                (TOK, D),
                lambda b, j: (
                    jnp.where(b * tpb + j < K_SC, K_SC, b * tpb + j), 0)),
            pl.BlockSpec((N_REC, D), lambda b, j: (0, 0)),
        ],
        out_specs=[
            pl.BlockSpec((1, tpb, TOK), lambda b, j: (b, 0, 0)),
            pl.BlockSpec((N_REC, 1, tpb, TOK),
                         lambda b, j: (0, b, 0, 0)),
        ],
        out_shape=[
            jax.ShapeDtypeStruct((B, tpb, TOK), jnp.float32),
            jax.ShapeDtypeStruct((N_REC, B, tpb, TOK), jnp.float32),
        ],
        compiler_params=pltpu.CompilerParams(
            dimension_semantics=("arbitrary", "arbitrary")),
    )(xf, W)


def _stitch(gate, logits, l0, l1, l2, gt, B, T):
    tpb = T // TOK

    def body(g_in, lg_in, l0_ref, l1_ref, l2_ref, gt_ref,
             gate_ref, logits_ref):
        c = pl.program_id(0)
        j = lax.rem(c, tpb)

        @pl.when(j == 0)
        def _():
            gate_ref[...] = g_in[...]
            logits_ref[...] = lg_in[...]

        gate_ref[0, pl.ds(j, 1), :] = gt_ref[...].reshape(1, TOK)
        logits_ref[:, 0, pl.ds(j, 1), :] = jnp.concatenate(
            [l0_ref[...].reshape(1, 1, TOK),
             l1_ref[...].reshape(1, 1, TOK),
             l2_ref[...].reshape(1, 1, TOK)], axis=0)

    return pl.pallas_call(
        body,
        grid=(K_SC,),
        in_specs=[
            pl.BlockSpec((1, tpb, TOK), lambda c: (c // tpb, 0, 0)),
            pl.BlockSpec((N_REC, 1, tpb, TOK),
                         lambda c: (0, c // tpb, 0, 0)),
            pl.BlockSpec((TOK,), lambda c: (c,)),
            pl.BlockSpec((TOK,), lambda c: (c,)),
            pl.BlockSpec((TOK,), lambda c: (c,)),
            pl.BlockSpec((TOK,), lambda c: (c,)),
        ],
        out_specs=[
            pl.BlockSpec((1, tpb, TOK), lambda c: (c // tpb, 0, 0)),
            pl.BlockSpec((N_REC, 1, tpb, TOK),
                         lambda c: (0, c // tpb, 0, 0)),
        ],
        out_shape=[
            jax.ShapeDtypeStruct((B, tpb, TOK), jnp.float32),
            jax.ShapeDtypeStruct((N_REC, B, tpb, TOK), jnp.float32),
        ],
        input_output_aliases={0: 0, 1: 1},
        compiler_params=pltpu.CompilerParams(
            dimension_semantics=("arbitrary",)),
    )(gate, logits, l0, l1, l2, gt)


def kernel(x, W):
    B, T, _ = x.shape
    N = B * T
    nsc = K_SC * TOK
    xf = x.reshape(N, D)
    l0, l1, l2, gt = _sc_router(xf, W, nsc)
    gate, logits_t = _tc_router(xf, W, B, T)
    gate, logits_t = _stitch(gate, logits_t, l0, l1, l2, gt, B, T)
    selected = jnp.broadcast_to(
        jnp.arange(T, dtype=jnp.int32)[None, :, None], (B, T, 1))
    return (selected, gate.reshape(B, T, 1),
            jnp.transpose(logits_t.reshape(N_REC, B, T), (1, 2, 0)))


# speedup vs baseline: 1.0020x; 1.0006x over previous
import functools

import jax
import jax.numpy as jnp
from jax import lax
from jax.experimental import pallas as pl
from jax.experimental.pallas import tpu as pltpu
from jax.experimental.pallas import tpu_sc as plsc

N_REC = 3
D = 768
NW = 32
TILE = 32
TB = 8
TOK = 2048
K_SC = 5


def _sc_router(xf, W, nsc):
    tps = nsc // NW
    nt = tps // TILE
    mesh = plsc.VectorSubcoreMesh(core_axis_name="c", subcore_axis_name="s")

    @functools.partial(
        pl.kernel,
        out_type=[jax.ShapeDtypeStruct((nsc,), jnp.float32)] * 4,
        mesh=mesh,
        scratch_types=[
            pltpu.VMEM((N_REC, D), jnp.float32),
            pltpu.VMEM((TILE, D), jnp.float32),
            pltpu.VMEM((TILE, D), jnp.float32),
            pltpu.VMEM((tps,), jnp.float32),
            pltpu.VMEM((tps,), jnp.float32),
            pltpu.VMEM((tps,), jnp.float32),
            pltpu.VMEM((tps,), jnp.float32),
            pltpu.SemaphoreType.DMA,
            pltpu.SemaphoreType.DMA,
        ],
        compiler_params=pltpu.CompilerParams(
            use_tc_tiling_on_sc=True, needs_layout_passes=False),
    )
    def k(x_hbm, w_hbm, o0_hbm, o1_hbm, o2_hbm, og_hbm,
          w_v, xb0, xb1, l0_v, l1_v, l2_v, gt_v, sem0, sem1):
        wid = lax.axis_index("s") * 2 + lax.axis_index("c")
        base = wid * tps
        pltpu.sync_copy(w_hbm, w_v)
        iota = lax.broadcasted_iota(jnp.int32, (16,), 0)

        def start_copy(t, buf, sem):
            pltpu.async_copy(
                x_hbm.at[pl.ds(base + t * TILE, TILE), :], buf, sem)

        def wait_copy(buf, sem):
            pltpu.make_async_copy(
                x_hbm.at[pl.ds(base, TILE), :], buf, sem).wait()

        def compute_tile(t, buf):
            zeros16 = jnp.zeros((16,), jnp.float32)

            @pl.loop(0, TILE // 16)
            def _(g):
                logit_vecs = [zeros16, zeros16, zeros16]
                for blk in range(16 // TB):

                    @pl.loop(0, D // 16,
                             init_carry=(zeros16,) * (3 * TB), unroll=2)
                    def accs(c, carry):
                        acc = list(carry)
                        c16 = c * 16
                        w0 = w_v[0, pl.ds(c16, 16)]
                        w1 = w_v[1, pl.ds(c16, 16)]
                        w2 = w_v[2, pl.ds(c16, 16)]
                        tokb = g * 16 + blk * TB
                        for tt in range(TB):
                            xv = buf[tokb + tt, pl.ds(c16, 16)]
                            acc[3 * tt] = acc[3 * tt] + xv * w0
                            acc[3 * tt + 1] = acc[3 * tt + 1] + xv * w1
                            acc[3 * tt + 2] = acc[3 * tt + 2] + xv * w2
                        return tuple(acc)

                    for tt in range(TB):
                        mask = iota == (blk * TB + tt)
                        for n in range(3):
                            s = jnp.sum(accs[3 * tt + n])
                            logit_vecs[n] = jnp.where(
                                mask, s, logit_vecs[n])

                l0, l1, l2 = logit_vecs
                m = jnp.maximum(jnp.maximum(l0, l1), l2)
                ssum = jnp.exp(l0 - m) + jnp.exp(l1 - m) + jnp.exp(l2 - m)
                off = t * TILE + g * 16
                gt_v[pl.ds(off, 16)] = 1.0 / ssum
                l0_v[pl.ds(off, 16)] = l0
                l1_v[pl.ds(off, 16)] = l1
                l2_v[pl.ds(off, 16)] = l2

        start_copy(0, xb0, sem0)
        start_copy(1, xb1, sem1)

        @pl.loop(0, nt, step=2)
        def _(t):
            wait_copy(xb0, sem0)
            compute_tile(t, xb0)

            @pl.when(t + 2 < nt)
            def _():
                start_copy(t + 2, xb0, sem0)

            wait_copy(xb1, sem1)
            compute_tile(t + 1, xb1)

            @pl.when(t + 3 < nt)
            def _():
                start_copy(t + 3, xb1, sem1)

        pltpu.sync_copy(l0_v, o0_hbm.at[pl.ds(base, tps)])
        pltpu.sync_copy(l1_v, o1_hbm.at[pl.ds(base, tps)])
        pltpu.sync_copy(l2_v, o2_hbm.at[pl.ds(base, tps)])
        pltpu.sync_copy(gt_v, og_hbm.at[pl.ds(base, tps)])

    return k(xf, W)


def _tc_router(xf, W, B, T):
    tpb = T // TOK

    def body(x_ref, w_ref, gate_ref, logits_ref):
        b = pl.program_id(0)
        j = pl.program_id(1)

        @pl.when(b * tpb + j >= K_SC)
        def _():
            logits = lax.dot_general(
                w_ref[...], x_ref[...], (((1,), (1,)), ((), ())),
                preferred_element_type=jnp.float32)
            m = jnp.max(logits, axis=0, keepdims=True)
            s = jnp.sum(jnp.exp(logits - m), axis=0, keepdims=True)
            logits_ref[:, 0, pl.ds(j, 1), :] = logits[:, None, :]
            gate_ref[0, pl.ds(j, 1), :] = (1.0 / s)

    return pl.pallas_call(
        body,
        grid=(B, tpb),
        in_specs=[
            pl.BlockSpec(
                (TOK, D),
                lambda b, j: (
                    jnp.where(b * tpb + j < K_SC, K_SC, b * tpb + j), 0)),
            pl.BlockSpec((N_REC, D), lambda b, j: (0, 0)),
        ],
        out_specs=[
            pl.BlockSpec((1, tpb, TOK), lambda b, j: (b, 0, 0)),
            pl.BlockSpec((N_REC, 1, tpb, TOK),
                         lambda b, j: (0, b, 0, 0)),
        ],
        out_shape=[
            jax.ShapeDtypeStruct((B, tpb, TOK), jnp.float32),
            jax.ShapeDtypeStruct((N_REC, B, tpb, TOK), jnp.float32),
        ],
        compiler_params=pltpu.CompilerParams(
            dimension_semantics=("arbitrary", "arbitrary")),
    )(xf, W)


def _stitch(gate, logits, l0, l1, l2, gt, B, T):
    tpb = T // TOK

    def body(g_in, lg_in, l0_ref, l1_ref, l2_ref, gt_ref,
             gate_ref, logits_ref):
        c = pl.program_id(0)
        j = lax.rem(c, tpb)

        @pl.when(j == 0)
        def _():
            gate_ref[...] = g_in[...]
            logits_ref[...] = lg_in[...]

        gate_ref[0, pl.ds(j, 1), :] = gt_ref[...].reshape(1, TOK)
        logits_ref[:, 0, pl.ds(j, 1), :] = jnp.concatenate(
            [l0_ref[...].reshape(1, 1, TOK),
             l1_ref[...].reshape(1, 1, TOK),
             l2_ref[...].reshape(1, 1, TOK)], axis=0)

    return pl.pallas_call(
        body,
        grid=(K_SC,),
        in_specs=[
            pl.BlockSpec((1, tpb, TOK), lambda c: (c // tpb, 0, 0)),
            pl.BlockSpec((N_REC, 1, tpb, TOK),
                         lambda c: (0, c // tpb, 0, 0)),
            pl.BlockSpec((TOK,), lambda c: (c,)),
            pl.BlockSpec((TOK,), lambda c: (c,)),
            pl.BlockSpec((TOK,), lambda c: (c,)),
            pl.BlockSpec((TOK,), lambda c: (c,)),
        ],
        out_specs=[
            pl.BlockSpec((1, tpb, TOK), lambda c: (c // tpb, 0, 0)),
            pl.BlockSpec((N_REC, 1, tpb, TOK),
                         lambda c: (0, c // tpb, 0, 0)),
        ],
        out_shape=[
            jax.ShapeDtypeStruct((B, tpb, TOK), jnp.float32),
            jax.ShapeDtypeStruct((N_REC, B, tpb, TOK), jnp.float32),
        ],
        input_output_aliases={0: 0, 1: 1},
        compiler_params=pltpu.CompilerParams(
            dimension_semantics=("arbitrary",)),
    )(gate, logits, l0, l1, l2, gt)


def kernel(x, W):
    B, T, _ = x.shape
    N = B * T
    nsc = K_SC * TOK
    xf = x.reshape(N, D)
    l0, l1, l2, gt = _sc_router(xf, W, nsc)
    gate, logits_t = _tc_router(xf, W, B, T)
    gate, logits_t = _stitch(gate, logits_t, l0, l1, l2, gt, B, T)
    selected = jnp.broadcast_to(
        jnp.arange(T, dtype=jnp.int32)[None, :, None], (B, T, 1))
    return (selected, gate.reshape(B, T, 1),
            jnp.transpose(logits_t.reshape(N_REC, B, T), (1, 2, 0)))
